# Initial kernel scaffold; baseline (speedup 1.0000x reference)
import jax
import jax.numpy as jnp
from jax.experimental import pallas as pl


def kernel(x, W_proj, b_proj, W_merge, b_merge, alpha, beta):
    raise NotImplementedError("write your pallas kernel here")


# speedup vs baseline: 2.6307x; 2.6307x over previous
import jax
import jax.numpy as jnp
from jax import lax
from jax.experimental import pallas as pl

_N, _IN, _HID, _FC, _CS, _FS, _H, _W = 2, 96, 96, 4, 8, 4, 224, 224
_TS = _H // _FS
_L = _TS * _TS
_S = _CS * _CS
_CG = 2 * _HID // _FC
_C2 = _CG // 2
_NT = _N * _FS * _FS

_HI = lax.Precision.HIGHEST


def _cluster_kernel(x_ref, wp_ref, bp_ref, wm_ref, bm_ref, ab_ref, pool_ref,
                    out_ref):
    X = x_ref[0]
    Wp = wp_ref[...]
    bp = bp_ref[...]
    proj = jnp.dot(X, Wp) + bp
    P = pool_ref[...]
    centers = lax.dot_general(P, proj, (((0,), (0,)), ((), ())),
                              precision=_HI)
    alpha = ab_ref[0, 0]
    beta = ab_ref[0, 1]
    idx = lax.broadcasted_iota(jnp.int32, (_L, _S), 1)

    groups = []
    for g in range(_FC):
        xg = proj[:, g * _CG:(g + 1) * _CG]
        cg = centers[:, g * _CG:(g + 1) * _CG]
        xp = xg[:, :_C2]
        xv = xg[:, _C2:]
        cp = cg[:, :_C2]
        cv = cg[:, _C2:]
        nx = xp / jnp.maximum(
            jnp.sqrt(jnp.sum(xp * xp, axis=1, keepdims=True)), 1e-12)
        nc = cp / jnp.maximum(
            jnp.sqrt(jnp.sum(cp * cp, axis=1, keepdims=True)), 1e-12)
        sim = lax.dot_general(nx, nc, (((1,), (1,)), ((), ())))
        sim = jax.nn.sigmoid(alpha * sim + beta)
        vmax = jnp.max(sim, axis=1, keepdims=True)
        cand = jnp.where(sim == vmax, idx, _S)
        imax = jnp.min(cand, axis=1, keepdims=True)
        wa = jnp.where(idx == imax, vmax, 0.0)
        ones = jnp.ones((_L, 1), dtype=jnp.float32)
        xv1 = jnp.concatenate([xv, ones], axis=1)
        sums = lax.dot_general(wa, xv1, (((0,), (0,)), ((), ())),
                               precision=_HI)
        num = cv + sums[:, :_C2]
        den = 1.0 + sums[:, _C2:_C2 + 1]
        newc = num / den
        groups.append(jnp.dot(wa, newc, precision=_HI))

    newx = jnp.concatenate(groups, axis=1)
    Wm = wm_ref[...]
    bm = bm_ref[...]
    out_ref[0] = jnp.dot(newx, Wm, precision=_HI) + bm


def _build_pool():
    h = jnp.arange(_TS) // (_TS // _CS)
    s_of_token = (h[:, None] * _CS + h[None, :]).reshape(_L)
    P = (s_of_token[:, None] == jnp.arange(_S)[None, :])
    return P.astype(jnp.float32) / float((_TS // _CS) ** 2)


@jax.jit
def kernel(x, W_proj, b_proj, W_merge, b_merge, alpha, beta):
    x_t = (x.reshape(_N, _IN, _FS, _TS, _FS, _TS)
           .transpose(0, 2, 4, 3, 5, 1)
           .reshape(_NT, _L, _IN))
    Wp = W_proj.T
    bp = b_proj.reshape(1, 2 * _HID)
    Wm = W_merge.T
    bm = b_merge.reshape(1, _IN)
    ab = jnp.concatenate([alpha, beta]).reshape(1, 2)
    P = _build_pool()

    out_t = pl.pallas_call(
        _cluster_kernel,
        grid=(_NT,),
        in_specs=[
            pl.BlockSpec((1, _L, _IN), lambda i: (i, 0, 0)),
            pl.BlockSpec((_IN, 2 * _HID), lambda i: (0, 0)),
            pl.BlockSpec((1, 2 * _HID), lambda i: (0, 0)),
            pl.BlockSpec((_IN, _IN), lambda i: (0, 0)),
            pl.BlockSpec((1, _IN), lambda i: (0, 0)),
            pl.BlockSpec((1, 2), lambda i: (0, 0)),
            pl.BlockSpec((_L, _S), lambda i: (0, 0)),
        ],
        out_specs=pl.BlockSpec((1, _L, _IN), lambda i: (i, 0, 0)),
        out_shape=jax.ShapeDtypeStruct((_NT, _L, _IN), jnp.float32),
    )(x_t, Wp, bp, Wm, bm, ab, P)

    return (out_t.reshape(_N, _FS, _FS, _TS, _TS, _IN)
            .transpose(0, 5, 1, 3, 2, 4)
            .reshape(_N, _IN, _H, _W))


# speedup vs baseline: 5.9123x; 2.2474x over previous
import jax
import jax.numpy as jnp
from jax import lax
from jax.experimental import pallas as pl

_N, _IN, _HID, _FC, _CS, _FS, _H, _W = 2, 96, 96, 4, 8, 4, 224, 224
_TS = _H // _FS
_L = _TS * _TS
_S = _CS * _CS
_CG = 2 * _HID // _FC
_C2 = _CG // 2
_PB = _TS // _CS


def _cluster_kernel(x_ref, wp_ref, bp_ref, wm_ref, bm_ref, ab_ref, out_ref):
    X3 = x_ref[0]
    rows = [jnp.transpose(X3[:, h, :]).reshape(_FS, _TS, _IN)
            for h in range(_TS)]
    Xt = jnp.stack(rows, axis=0)
    Xt = Xt.reshape(_TS * _H, _IN)

    Wp = wp_ref[...]
    bp = bp_ref[...]
    proj = jnp.dot(Xt, Wp) + bp
    proj = proj.reshape(_TS, _FS, _TS, 2 * _HID)

    alpha = ab_ref[0, 0]
    beta = ab_ref[0, 1]
    Wm = wm_ref[...]
    bm = bm_ref[...]
    idx = lax.broadcasted_iota(jnp.int32, (_L, _S), 1)

    outs = []
    for fw in range(_FS):
        pw = proj[:, fw].reshape(_L, 2 * _HID)
        cw = pw.reshape(_CS, _PB, _CS, _PB, 2 * _HID)
        cw = jnp.sum(cw, axis=(1, 3)).reshape(_S, 2 * _HID) / float(_PB * _PB)
        groups = []
        for g in range(_FC):
            xg = pw[:, g * _CG:(g + 1) * _CG]
            cg = cw[:, g * _CG:(g + 1) * _CG]
            xp = xg[:, :_C2]
            xv = xg[:, _C2:]
            cp = cg[:, :_C2]
            cv = cg[:, _C2:]
            nx = xp / jnp.maximum(
                jnp.sqrt(jnp.sum(xp * xp, axis=1, keepdims=True)), 1e-12)
            nc = cp / jnp.maximum(
                jnp.sqrt(jnp.sum(cp * cp, axis=1, keepdims=True)), 1e-12)
            sim = lax.dot_general(nx, nc, (((1,), (1,)), ((), ())))
            t = alpha * sim + beta
            tmax = jnp.max(t, axis=1, keepdims=True)
            cand = jnp.where(t == tmax, idx, _S)
            imax = jnp.min(cand, axis=1, keepdims=True)
            vmax = jax.nn.sigmoid(tmax)
            wa = jnp.where(idx == imax, vmax, 0.0)
            ones = jnp.ones((_L, 1), dtype=jnp.float32)
            xv1 = jnp.concatenate([xv, ones], axis=1)
            sums = lax.dot_general(wa, xv1, (((0,), (0,)), ((), ())))
            num = cv + sums[:, :_C2]
            den = 1.0 + sums[:, _C2:_C2 + 1]
            newc = num / den
            groups.append(jnp.dot(wa, newc))
        newx = jnp.concatenate(groups, axis=1)
        outs.append((jnp.dot(newx, Wm) + bm).reshape(_TS, _TS, _IN))
    out_t = jnp.stack(outs, axis=0)
    for h in range(_TS):
        row = out_t[:, h].reshape(_H, _IN)
        out_ref[0, :, h, :] = jnp.transpose(row)


def kernel(x, W_proj, b_proj, W_merge, b_merge, alpha, beta):
    Wp = W_proj.T
    bp = b_proj.reshape(1, 2 * _HID)
    Wm = W_merge.T
    bm = b_merge.reshape(1, _IN)
    ab = jnp.concatenate([alpha, beta]).reshape(1, 2)

    return pl.pallas_call(
        _cluster_kernel,
        grid=(_N, _FS),
        in_specs=[
            pl.BlockSpec((1, _IN, _TS, _W), lambda n, fh: (n, 0, fh, 0)),
            pl.BlockSpec((_IN, 2 * _HID), lambda n, fh: (0, 0)),
            pl.BlockSpec((1, 2 * _HID), lambda n, fh: (0, 0)),
            pl.BlockSpec((_IN, _IN), lambda n, fh: (0, 0)),
            pl.BlockSpec((1, _IN), lambda n, fh: (0, 0)),
            pl.BlockSpec((1, 2), lambda n, fh: (0, 0)),
        ],
        out_specs=pl.BlockSpec((1, _IN, _TS, _W), lambda n, fh: (n, 0, fh, 0)),
        out_shape=jax.ShapeDtypeStruct((_N, _IN, _H, _W), jnp.float32),
    )(x, Wp, bp, Wm, bm, ab)


# speedup vs baseline: 7.6421x; 1.2926x over previous
import jax
import jax.numpy as jnp
from jax import lax
from jax.experimental import pallas as pl

_N, _IN, _HID, _FC, _CS, _FS, _H, _W = 2, 96, 96, 4, 8, 4, 224, 224
_TS = _H // _FS
_L = _TS * _TS
_S = _CS * _CS
_CG = 2 * _HID // _FC
_C2 = _CG // 2
_PB = _TS // _CS


def _cluster_kernel(x_ref, wp_ref, bp_ref, wm_ref, bm_ref, ab_ref, out_ref):
    X3 = x_ref[0]
    rows = [jnp.transpose(X3[:, h, :]).reshape(_FS, _TS, _IN)
            for h in range(_TS)]
    Xt = jnp.stack(rows, axis=0)
    Xt = Xt.reshape(_TS * _H, _IN)

    Wp = wp_ref[...]
    bp = bp_ref[...]
    proj = jnp.dot(Xt, Wp) + bp
    proj = proj.reshape(_TS, _FS, _TS, 2 * _HID)

    alpha = ab_ref[0, 0]
    beta = ab_ref[0, 1]
    Wm = wm_ref[...]
    bm = bm_ref[...]
    idx = lax.broadcasted_iota(jnp.int32, (_S, _L), 0)

    outs = []
    for fw in range(_FS):
        pw = proj[:, fw].reshape(_L, 2 * _HID)
        cw = pw.reshape(_CS, _PB, _CS, _PB, 2 * _HID)
        cw = jnp.sum(cw, axis=(1, 3)).reshape(_S, 2 * _HID) / float(_PB * _PB)
        groups = []
        for g in range(_FC):
            xg = pw[:, g * _CG:(g + 1) * _CG]
            cg = cw[:, g * _CG:(g + 1) * _CG]
            xp = xg[:, :_C2]
            xv = xg[:, _C2:]
            cp = cg[:, :_C2]
            cv = cg[:, _C2:]
            nx = xp / jnp.maximum(
                jnp.sqrt(jnp.sum(xp * xp, axis=1, keepdims=True)), 1e-12)
            nc = cp / jnp.maximum(
                jnp.sqrt(jnp.sum(cp * cp, axis=1, keepdims=True)), 1e-12)
            sim = lax.dot_general(nc, nx, (((1,), (1,)), ((), ())))
            t = alpha * sim + beta
            tmax = jnp.max(t, axis=0, keepdims=True)
            cand = jnp.where(t == tmax, idx, _S)
            imax = jnp.min(cand, axis=0, keepdims=True)
            vmax = jax.nn.sigmoid(tmax)
            wa = jnp.where(idx == imax, vmax, 0.0)
            ones = jnp.ones((_L, 1), dtype=jnp.float32)
            xv1 = jnp.concatenate([xv, ones], axis=1)
            sums = lax.dot_general(wa, xv1, (((1,), (0,)), ((), ())))
            num = cv + sums[:, :_C2]
            den = 1.0 + sums[:, _C2:_C2 + 1]
            newc = num / den
            groups.append(
                lax.dot_general(wa, newc, (((0,), (0,)), ((), ()))))
        newx = jnp.concatenate(groups, axis=1)
        outs.append((jnp.dot(newx, Wm) + bm).reshape(_TS, _TS, _IN))
    out_t = jnp.stack(outs, axis=0)
    for h in range(_TS):
        row = out_t[:, h].reshape(_H, _IN)
        out_ref[0, :, h, :] = jnp.transpose(row)


def kernel(x, W_proj, b_proj, W_merge, b_merge, alpha, beta):
    Wp = W_proj.T
    bp = b_proj.reshape(1, 2 * _HID)
    Wm = W_merge.T
    bm = b_merge.reshape(1, _IN)
    ab = jnp.concatenate([alpha, beta]).reshape(1, 2)

    return pl.pallas_call(
        _cluster_kernel,
        grid=(_N, _FS),
        in_specs=[
            pl.BlockSpec((1, _IN, _TS, _W), lambda n, fh: (n, 0, fh, 0)),
            pl.BlockSpec((_IN, 2 * _HID), lambda n, fh: (0, 0)),
            pl.BlockSpec((1, 2 * _HID), lambda n, fh: (0, 0)),
            pl.BlockSpec((_IN, _IN), lambda n, fh: (0, 0)),
            pl.BlockSpec((1, _IN), lambda n, fh: (0, 0)),
            pl.BlockSpec((1, 2), lambda n, fh: (0, 0)),
        ],
        out_specs=pl.BlockSpec((1, _IN, _TS, _W), lambda n, fh: (n, 0, fh, 0)),
        out_shape=jax.ShapeDtypeStruct((_N, _IN, _H, _W), jnp.float32),
    )(x, Wp, bp, Wm, bm, ab)
